# Initial kernel scaffold; baseline (speedup 1.0000x reference)
import jax
import jax.numpy as jnp
from jax.experimental import pallas as pl


def kernel(transition_counts, pseudocount, hidden_states):
    raise NotImplementedError("write your pallas kernel here")


# speedup vs baseline: 2.9140x; 2.9140x over previous
import jax
import jax.numpy as jnp
from jax.experimental import pallas as pl
from jax.experimental.pallas import tpu as pltpu

V = 32
C = 128


def _normalize_block(pc_ref, counts_ref, out_ref, rs_ref):
    x = counts_ref[0] + pc_ref[0, 0]
    rs = x.sum(axis=(0, 2))
    denom = jnp.where(rs > 0, rs, jnp.float32(1.0))
    recip = jnp.float32(1.0) / denom
    out_ref[0] = x * recip[None, :, None]
    rs_ref[0, 0] = rs


def kernel(transition_counts, pseudocount, hidden_states):
    del hidden_states
    counts = transition_counts.reshape(V, V, C, C)
    pc = jnp.asarray(pseudocount, jnp.float32).reshape(1, 1)
    out, rs = pl.pallas_call(
        _normalize_block,
        grid=(V,),
        in_specs=[
            pl.BlockSpec(memory_space=pltpu.SMEM),
            pl.BlockSpec((1, V, C, C), lambda i: (i, 0, 0, 0)),
        ],
        out_specs=[
            pl.BlockSpec((1, V, C, C), lambda i: (i, 0, 0, 0)),
            pl.BlockSpec((1, 1, C), lambda i: (i, 0, 0)),
        ],
        out_shape=[
            jax.ShapeDtypeStruct((V, V, C, C), jnp.float32),
            jax.ShapeDtypeStruct((V, 1, C), jnp.float32),
        ],
    )(pc, counts)
    return out.reshape(-1), rs.reshape(-1)
